# Initial kernel scaffold; baseline (speedup 1.0000x reference)
import jax
import jax.numpy as jnp
from jax.experimental import pallas as pl


def kernel(x, edge_index, edge_type, edge_vector, Wt, bt, Wk, bk, Wq, bq, Wv, bv, ln_g, ln_b):
    raise NotImplementedError("write your pallas kernel here")


# speedup vs baseline: 2.8566x; 2.8566x over previous
import functools

import jax
import jax.numpy as jnp
from jax import lax
from jax.experimental import pallas as pl
from jax.experimental.pallas import tpu as pltpu
from jax.experimental.pallas import tpu_sc as plsc

N, E, D, DE = 10000, 160000, 256, 16
NP = 10240
EP = 163840
BN = 640
BE = 640
GCH = 64
SCH = 128
EPW_G = EP // 32
EPW_S = EP // 16
NH = NP // 2

_mesh = plsc.VectorSubcoreMesh(core_axis_name="c", subcore_axis_name="s")
_sc_params = pltpu.CompilerParams(needs_layout_passes=False)


def _gelu(v):
    return 0.5 * v * (1.0 + lax.erf(v * 0.7071067811865476))


def _node_body(x_ref, aux_ref, wt1t_ref, wqt_ref, wk_ref, t_ref, qk_ref, bkq_ref):
    x = x_ref[...]
    ln_g = aux_ref[0:1, :]
    ln_b = aux_ref[1:2, :]
    bq = aux_ref[2:3, :]
    bk = aux_ref[3:4, :]
    mu = jnp.mean(x, axis=1, keepdims=True)
    xc = x - mu
    var = jnp.mean(xc * xc, axis=1, keepdims=True)
    xn = xc * lax.rsqrt(var + 1e-5) * ln_g + ln_b
    t = jnp.dot(xn, wt1t_ref[...], preferred_element_type=jnp.float32)
    q = jnp.dot(xn, wqt_ref[...], preferred_element_type=jnp.float32) + bq
    qk = jnp.dot(q, wk_ref[...], preferred_element_type=jnp.float32)
    t_ref[...] = t
    qk_ref[...] = qk
    bkq_ref[...] = jnp.sum(q * bk, axis=1, keepdims=True)


def _node_stage(x, aux, wt1t, wqt, wk):
    return pl.pallas_call(
        _node_body,
        grid=(NP // BN,),
        in_specs=[
            pl.BlockSpec((BN, D), lambda i: (i, 0)),
            pl.BlockSpec((8, D), lambda i: (0, 0)),
            pl.BlockSpec((D, D), lambda i: (0, 0)),
            pl.BlockSpec((D, D), lambda i: (0, 0)),
            pl.BlockSpec((D, D), lambda i: (0, 0)),
        ],
        out_specs=[
            pl.BlockSpec((BN, D), lambda i: (i, 0)),
            pl.BlockSpec((BN, D), lambda i: (i, 0)),
            pl.BlockSpec((BN, 1), lambda i: (i, 0)),
        ],
        out_shape=[
            jax.ShapeDtypeStruct((NP, D), jnp.float32),
            jax.ShapeDtypeStruct((NP, D), jnp.float32),
            jax.ShapeDtypeStruct((NP, 1), jnp.float32),
        ],
    )(x, aux, wt1t, wqt, wk)


@functools.partial(
    pl.kernel,
    mesh=_mesh,
    out_type=[
        jax.ShapeDtypeStruct((EP, D), jnp.float32),
        jax.ShapeDtypeStruct((EP, D), jnp.float32),
        jax.ShapeDtypeStruct((EP,), jnp.float32),
    ],
    scratch_types=[
        pltpu.VMEM((GCH,), jnp.int32),
        pltpu.VMEM((GCH,), jnp.int32),
        pltpu.VMEM((GCH, D), jnp.float32),
        pltpu.VMEM((GCH, D), jnp.float32),
        pltpu.VMEM((NP,), jnp.float32),
        pltpu.VMEM((GCH,), jnp.float32),
        pltpu.SemaphoreType.DMA,
        pltpu.SemaphoreType.DMA,
    ],
    compiler_params=_sc_params,
)
def _gather_stage(tnode_hbm, qk_hbm, bkq_hbm, src_hbm, dst_hbm,
                  ts_hbm, qd_hbm, bkqd_hbm,
                  si_v, di_v, tr_v, qr_v, bkq_v, bko_v, sem1, sem2):
    c = lax.axis_index("c")
    s = lax.axis_index("s")
    wid = s * 2 + c
    pltpu.sync_copy(bkq_hbm, bkq_v)

    def body(g, carry):
        base = wid * EPW_G + g * GCH
        pltpu.sync_copy(src_hbm.at[pl.ds(base, GCH)], si_v)
        pltpu.sync_copy(dst_hbm.at[pl.ds(base, GCH)], di_v)
        cp1 = pltpu.async_copy(tnode_hbm.at[si_v], tr_v, sem1)
        cp2 = pltpu.async_copy(qk_hbm.at[di_v], qr_v, sem2)
        for j in range(GCH // 16):
            i16 = di_v[pl.ds(j * 16, 16)]
            bko_v[pl.ds(j * 16, 16)] = plsc.load_gather(bkq_v, [i16])
        cp1.wait()
        cp2.wait()
        pltpu.sync_copy(tr_v, ts_hbm.at[pl.ds(base, GCH)])
        pltpu.sync_copy(qr_v, qd_hbm.at[pl.ds(base, GCH)])
        pltpu.sync_copy(bko_v, bkqd_hbm.at[pl.ds(base, GCH)])
        return carry

    lax.fori_loop(0, EPW_G // GCH, body, 0)


def _edge_body(ts_ref, qd_ref, bkqd_ref, ev_ref, et_ref, wt2t_ref, aux_ref,
               m_ref, w_ref):
    bt = aux_ref[4:5, :]
    pre = ts_ref[...] + jnp.dot(ev_ref[...], wt2t_ref[...],
                                preferred_element_type=jnp.float32) + bt
    x_t = _gelu(pre)
    att = (jnp.sum(x_t * qd_ref[...], axis=1, keepdims=True)
           + bkqd_ref[...]) * et_ref[...] * 0.0625
    w = jnp.exp(att)
    m_ref[...] = x_t * w
    w_ref[...] = w


def _edge_stage(t_src, qk_dst, bkq_dst, ev, et, wt2t, aux):
    return pl.pallas_call(
        _edge_body,
        grid=(EP // BE,),
        in_specs=[
            pl.BlockSpec((BE, D), lambda i: (i, 0)),
            pl.BlockSpec((BE, D), lambda i: (i, 0)),
            pl.BlockSpec((BE, 1), lambda i: (i, 0)),
            pl.BlockSpec((BE, DE), lambda i: (i, 0)),
            pl.BlockSpec((BE, 1), lambda i: (i, 0)),
            pl.BlockSpec((DE, D), lambda i: (0, 0)),
            pl.BlockSpec((8, D), lambda i: (0, 0)),
        ],
        out_specs=[
            pl.BlockSpec((BE, D), lambda i: (i, 0)),
            pl.BlockSpec((BE, 1), lambda i: (i, 0)),
        ],
        out_shape=[
            jax.ShapeDtypeStruct((EP, D), jnp.float32),
            jax.ShapeDtypeStruct((EP, 1), jnp.float32),
        ],
    )(t_src, qk_dst, bkq_dst, ev, et, wt2t, aux)


@functools.partial(
    pl.kernel,
    mesh=_mesh,
    out_type=jax.ShapeDtypeStruct((NP, D), jnp.float32),
    scratch_types=[
        pltpu.VMEM_SHARED((NP, D // 2), jnp.float32),
        pltpu.VMEM((SCH, D // 2), jnp.float32),
        pltpu.VMEM((SCH,), jnp.int32),
    ],
    compiler_params=_sc_params,
)
def _scatter_stage(m_hbm, dst_hbm, zu_hbm, u_hbm, u_sh, buf, idx_v):
    c = lax.axis_index("c")
    s = lax.axis_index("s")
    half = D // 2

    @pl.when(s == 0)
    def _():
        pltpu.sync_copy(zu_hbm, u_sh)

    plsc.subcore_barrier()

    def body(g, carry):
        base = s * EPW_S + g * SCH
        pltpu.sync_copy(dst_hbm.at[pl.ds(base, SCH)], idx_v)
        pltpu.sync_copy(m_hbm.at[pl.ds(base, SCH), pl.ds(c * half, half)], buf)
        pltpu.sync_copy(buf, u_sh.at[idx_v], add=True)
        return carry

    lax.fori_loop(0, EPW_S // SCH, body, 0)
    plsc.subcore_barrier()

    rows = NP // 16
    r0 = s * rows
    pltpu.sync_copy(u_sh.at[pl.ds(r0, rows)],
                    u_hbm.at[pl.ds(r0, rows), pl.ds(c * half, half)])


@functools.partial(
    pl.kernel,
    mesh=_mesh,
    out_type=jax.ShapeDtypeStruct((NP, D), jnp.float32),
    scratch_types=[
        pltpu.VMEM_SHARED((NP, 128), jnp.float32),
        pltpu.VMEM((SCH,), jnp.float32),
        pltpu.VMEM((SCH,), jnp.int32),
        pltpu.VMEM((SCH, 128), jnp.float32),
    ],
    compiler_params=_sc_params,
)
def _denom_stage(w_hbm, dst_hbm, zu_hbm, den_hbm, wtab_sh, wbuf, idx_v, wrow_v):
    c = lax.axis_index("c")
    s = lax.axis_index("s")

    @pl.when(s == 0)
    def _():
        pltpu.sync_copy(zu_hbm, wtab_sh)

    pltpu.sync_copy(zu_hbm.at[pl.ds(0, SCH)], wrow_v)
    plsc.subcore_barrier()

    zeros16 = jnp.zeros((16,), jnp.int32)
    iota16 = lax.iota(jnp.int32, 16)
    epw = EP // 32

    def body(g, carry):
        base = (c * 16 + s) * epw + g * SCH
        pltpu.sync_copy(dst_hbm.at[pl.ds(base, SCH)], idx_v)
        pltpu.sync_copy(w_hbm.at[pl.ds(base, SCH)], wbuf)
        for j in range(SCH // 16):
            w16 = wbuf[pl.ds(j * 16, 16)]
            plsc.store_scatter(wrow_v, [iota16 + j * 16, zeros16], w16)
        pltpu.sync_copy(wrow_v, wtab_sh.at[idx_v], add=True)
        return carry

    lax.fori_loop(0, epw // SCH, body, 0)
    plsc.subcore_barrier()

    rows = NP // 16
    r0 = s * rows
    pltpu.sync_copy(wtab_sh.at[pl.ds(r0, rows)],
                    den_hbm.at[pl.ds(r0, rows), pl.ds(c * 128, 128)])


def _final_body(x_ref, u_ref, den_ref, wvt_ref, aux_ref, out_ref):
    bv = aux_ref[5:6, :]
    den = den_ref[:, 0:1] + den_ref[:, 128:129]
    inv = 1.0 / (den + 1e-16)
    aggr = jnp.dot(u_ref[...] * inv, wvt_ref[...],
                   preferred_element_type=jnp.float32) + bv * (den * inv)
    out_ref[...] = x_ref[...] + _gelu(aggr)


def _final_stage(x, u, den, wvt, aux):
    return pl.pallas_call(
        _final_body,
        grid=(N // 400,),
        in_specs=[
            pl.BlockSpec((400, D), lambda i: (i, 0)),
            pl.BlockSpec((400, D), lambda i: (i, 0)),
            pl.BlockSpec((400, D), lambda i: (i, 0)),
            pl.BlockSpec((D, D), lambda i: (0, 0)),
            pl.BlockSpec((8, D), lambda i: (0, 0)),
        ],
        out_specs=pl.BlockSpec((400, D), lambda i: (i, 0)),
        out_shape=jax.ShapeDtypeStruct((N, D), jnp.float32),
    )(x, u, den, wvt, aux)


def kernel(x, edge_index, edge_type, edge_vector, Wt, bt, Wk, bk, Wq, bq,
           Wv, bv, ln_g, ln_b):
    f32 = jnp.float32
    src = edge_index[0]
    dst = edge_index[1]
    pad_e = EP - E
    pad_idx = jnp.full((pad_e,), N, jnp.int32)
    srcp = jnp.concatenate([src, pad_idx])
    dstp = jnp.concatenate([dst, pad_idx])
    evp = jnp.concatenate([edge_vector, jnp.zeros((pad_e, DE), f32)])
    etp = jnp.concatenate([edge_type, jnp.zeros((pad_e, 1), f32)])
    xp = jnp.concatenate([x, jnp.zeros((NP - N, D), f32)])

    zero = jnp.zeros((D,), f32)
    aux = jnp.stack([ln_g, ln_b, bq, bk, bt, bv, zero, zero])
    wt1t = Wt[:, :D].T
    wt2t = Wt[:, D:].T

    t_node, qk, bkq = _node_stage(xp, aux, wt1t, Wq.T, Wk)
    t_src, qk_dst, bkq_dst = _gather_stage(t_node, qk, bkq.reshape(NP),
                                           srcp, dstp)
    m, w = _edge_stage(t_src, qk_dst, bkq_dst.reshape(EP, 1), evp, etp,
                       wt2t, aux)
    zu = jnp.zeros((NP, 128), f32)
    u = _scatter_stage(m, dstp, zu)
    den = _denom_stage(w.reshape(EP), dstp, zu)
    return _final_stage(x, u[:N], den[:N], Wv.T, aux)


# speedup vs baseline: 3.2187x; 1.1268x over previous
import functools

import jax
import jax.numpy as jnp
from jax import lax
from jax.experimental import pallas as pl
from jax.experimental.pallas import tpu as pltpu
from jax.experimental.pallas import tpu_sc as plsc

N, E, D, DE = 10000, 160000, 256, 16
NP = 10240
EP = 163840
BN = 640
BE = 640
GCH = 64
SCH = 128
EPW_G = EP // 32
EPW_S = EP // 16
NH = NP // 2

_mesh = plsc.VectorSubcoreMesh(core_axis_name="c", subcore_axis_name="s")
_sc_params = pltpu.CompilerParams(needs_layout_passes=False)


def _gelu(v):
    return 0.5 * v * (1.0 + lax.erf(v * 0.7071067811865476))


def _node_body(x_ref, aux_ref, wt1t_ref, wqt_ref, wk_ref, t_ref, qk_ref, bkq_ref):
    x = x_ref[...]
    ln_g = aux_ref[0:1, :]
    ln_b = aux_ref[1:2, :]
    bq = aux_ref[2:3, :]
    bk = aux_ref[3:4, :]
    mu = jnp.mean(x, axis=1, keepdims=True)
    xc = x - mu
    var = jnp.mean(xc * xc, axis=1, keepdims=True)
    xn = xc * lax.rsqrt(var + 1e-5) * ln_g + ln_b
    t = jnp.dot(xn, wt1t_ref[...], preferred_element_type=jnp.float32)
    q = jnp.dot(xn, wqt_ref[...], preferred_element_type=jnp.float32) + bq
    qk = jnp.dot(q, wk_ref[...], preferred_element_type=jnp.float32)
    t_ref[...] = t
    qk_ref[...] = qk
    bkq_ref[...] = jnp.sum(q * bk, axis=1, keepdims=True)


def _node_stage(x, aux, wt1t, wqt, wk):
    return pl.pallas_call(
        _node_body,
        grid=(NP // BN,),
        in_specs=[
            pl.BlockSpec((BN, D), lambda i: (i, 0)),
            pl.BlockSpec((8, D), lambda i: (0, 0)),
            pl.BlockSpec((D, D), lambda i: (0, 0)),
            pl.BlockSpec((D, D), lambda i: (0, 0)),
            pl.BlockSpec((D, D), lambda i: (0, 0)),
        ],
        out_specs=[
            pl.BlockSpec((BN, D), lambda i: (i, 0)),
            pl.BlockSpec((BN, D), lambda i: (i, 0)),
            pl.BlockSpec((BN, 1), lambda i: (i, 0)),
        ],
        out_shape=[
            jax.ShapeDtypeStruct((NP, D), jnp.float32),
            jax.ShapeDtypeStruct((NP, D), jnp.float32),
            jax.ShapeDtypeStruct((NP, 1), jnp.float32),
        ],
    )(x, aux, wt1t, wqt, wk)


@functools.partial(
    pl.kernel,
    mesh=_mesh,
    out_type=[
        jax.ShapeDtypeStruct((EP, D), jnp.float32),
        jax.ShapeDtypeStruct((EP, D), jnp.float32),
        jax.ShapeDtypeStruct((EP,), jnp.float32),
    ],
    scratch_types=[
        pltpu.VMEM((EPW_G,), jnp.int32),
        pltpu.VMEM((EPW_G,), jnp.int32),
        pltpu.VMEM((GCH, D), jnp.float32),
        pltpu.VMEM((GCH, D), jnp.float32),
        pltpu.VMEM((GCH, D), jnp.float32),
        pltpu.VMEM((GCH, D), jnp.float32),
        pltpu.VMEM((GCH,), jnp.float32),
        pltpu.VMEM((GCH,), jnp.float32),
        pltpu.VMEM((NP,), jnp.float32),
        pltpu.SemaphoreType.DMA,
        pltpu.SemaphoreType.DMA,
        pltpu.SemaphoreType.DMA,
        pltpu.SemaphoreType.DMA,
    ],
    compiler_params=_sc_params,
)
def _gather_stage(tnode_hbm, qk_hbm, bkq_hbm, src_hbm, dst_hbm,
                  ts_hbm, qd_hbm, bkqd_hbm,
                  si_all, di_all, tr0, tr1, qr0, qr1, bko0, bko1, bkq_v,
                  gs0, gs1, ws0, ws1):
    c = lax.axis_index("c")
    s = lax.axis_index("s")
    wid = s * 2 + c
    e0 = wid * EPW_G
    G = EPW_G // GCH
    trs = (tr0, tr1)
    qrs = (qr0, qr1)
    bkos = (bko0, bko1)
    gss = (gs0, gs1)
    wss = (ws0, ws1)

    pltpu.sync_copy(src_hbm.at[pl.ds(e0, EPW_G)], si_all)
    pltpu.sync_copy(dst_hbm.at[pl.ds(e0, EPW_G)], di_all)
    pltpu.sync_copy(bkq_hbm, bkq_v)

    def start_gather(g, a):
        off = g * GCH
        pltpu.async_copy(tnode_hbm.at[si_all.at[pl.ds(off, GCH)]],
                         trs[a], gss[a])
        pltpu.async_copy(qk_hbm.at[di_all.at[pl.ds(off, GCH)]],
                         qrs[a], gss[a])

    def wait_gather(a):
        pltpu.make_async_copy(ts_hbm.at[pl.ds(0, GCH)], trs[a], gss[a]).wait()
        pltpu.make_async_copy(ts_hbm.at[pl.ds(0, GCH)], qrs[a], gss[a]).wait()

    def wait_wb(a):
        pltpu.make_async_copy(ts_hbm.at[pl.ds(0, GCH)], trs[a], wss[a]).wait()
        pltpu.make_async_copy(ts_hbm.at[pl.ds(0, GCH)], qrs[a], wss[a]).wait()
        pltpu.make_async_copy(bkqd_hbm.at[pl.ds(0, GCH)], bkos[a],
                              wss[a]).wait()

    start_gather(0, 0)

    def substep(g, a, h, gate_ws, gate_next):
        b = 1 - a
        wait_gather(a)
        off = g * GCH
        for j in range(GCH // 16):
            i16 = di_all[pl.ds(off + j * 16, 16)]
            bkos[a][pl.ds(j * 16, 16)] = plsc.load_gather(bkq_v, [i16])
        if gate_ws is None:
            wait_wb(b)
        else:
            @pl.when(gate_ws)
            def _():
                wait_wb(b)
        if gate_next is None:
            start_gather(g + 1, b)
        else:
            @pl.when(gate_next)
            def _():
                start_gather(g + 1, b)
        pltpu.async_copy(trs[a], ts_hbm.at[pl.ds(e0 + off, GCH)], wss[a])
        pltpu.async_copy(qrs[a], qd_hbm.at[pl.ds(e0 + off, GCH)], wss[a])
        pltpu.async_copy(bkos[a], bkqd_hbm.at[pl.ds(e0 + off, GCH)], wss[a])

    def body(h, carry):
        substep(2 * h, 0, h, h > 0, None)
        substep(2 * h + 1, 1, h, None, h < G // 2 - 1)
        return carry

    lax.fori_loop(0, G // 2, body, 0)
    wait_wb(1)


def _edge_body(ts_ref, qd_ref, bkqd_ref, ev_ref, et_ref, wt2t_ref, aux_ref,
               m_ref, w_ref):
    bt = aux_ref[4:5, :]
    pre = ts_ref[...] + jnp.dot(ev_ref[...], wt2t_ref[...],
                                preferred_element_type=jnp.float32) + bt
    x_t = _gelu(pre)
    att = (jnp.sum(x_t * qd_ref[...], axis=1, keepdims=True)
           + bkqd_ref[...]) * et_ref[...] * 0.0625
    w = jnp.exp(att)
    m_ref[...] = x_t * w
    w_ref[...] = w


def _edge_stage(t_src, qk_dst, bkq_dst, ev, et, wt2t, aux):
    return pl.pallas_call(
        _edge_body,
        grid=(EP // BE,),
        in_specs=[
            pl.BlockSpec((BE, D), lambda i: (i, 0)),
            pl.BlockSpec((BE, D), lambda i: (i, 0)),
            pl.BlockSpec((BE, 1), lambda i: (i, 0)),
            pl.BlockSpec((BE, DE), lambda i: (i, 0)),
            pl.BlockSpec((BE, 1), lambda i: (i, 0)),
            pl.BlockSpec((DE, D), lambda i: (0, 0)),
            pl.BlockSpec((8, D), lambda i: (0, 0)),
        ],
        out_specs=[
            pl.BlockSpec((BE, D), lambda i: (i, 0)),
            pl.BlockSpec((BE, 1), lambda i: (i, 0)),
        ],
        out_shape=[
            jax.ShapeDtypeStruct((EP, D), jnp.float32),
            jax.ShapeDtypeStruct((EP, 1), jnp.float32),
        ],
    )(t_src, qk_dst, bkq_dst, ev, et, wt2t, aux)


@functools.partial(
    pl.kernel,
    mesh=_mesh,
    out_type=jax.ShapeDtypeStruct((NP, D), jnp.float32),
    scratch_types=[
        pltpu.VMEM_SHARED((NP, D // 2), jnp.float32),
        pltpu.VMEM((EPW_S,), jnp.int32),
        pltpu.VMEM((SCH, D // 2), jnp.float32),
        pltpu.VMEM((SCH, D // 2), jnp.float32),
        pltpu.VMEM((SCH,), jnp.int32),
        pltpu.VMEM((SCH,), jnp.int32),
        pltpu.SemaphoreType.DMA,
        pltpu.SemaphoreType.DMA,
    ],
    compiler_params=_sc_params,
)
def _scatter_stage(m_hbm, dst_hbm, zu_hbm, u_hbm, u_sh, di_all,
                   buf0, buf1, idx0, idx1, ms0, ms1):
    c = lax.axis_index("c")
    s = lax.axis_index("s")
    half = D // 2
    e0 = s * EPW_S
    G = EPW_S // SCH
    bufs = (buf0, buf1)
    idxs = (idx0, idx1)
    mss = (ms0, ms1)

    @pl.when(s == 0)
    def _():
        pltpu.sync_copy(zu_hbm, u_sh)

    pltpu.sync_copy(dst_hbm.at[pl.ds(e0, EPW_S)], di_all)
    plsc.subcore_barrier()

    def start_read(g, a):
        pltpu.async_copy(
            m_hbm.at[pl.ds(e0 + g * SCH, SCH), pl.ds(c * half, half)],
            bufs[a], mss[a])

    def wait_read(a):
        pltpu.make_async_copy(
            m_hbm.at[pl.ds(0, SCH), pl.ds(0, half)], bufs[a], mss[a]).wait()

    def build_idx(g, a):
        off = g * SCH
        for j in range(SCH // 16):
            idxs[a][pl.ds(j * 16, 16)] = di_all[pl.ds(off + j * 16, 16)]

    start_read(0, 0)
    build_idx(0, 0)

    def substep(g, a, gate_next):
        b = 1 - a
        if gate_next is None:
            start_read(g + 1, b)
            build_idx(g + 1, b)
        else:
            @pl.when(gate_next)
            def _():
                start_read(g + 1, b)
                build_idx(g + 1, b)
        wait_read(a)
        pltpu.sync_copy(bufs[a], u_sh.at[idxs[a]], add=True)

    def body(h, carry):
        substep(2 * h, 0, None)
        substep(2 * h + 1, 1, h < G // 2 - 1)
        return carry

    lax.fori_loop(0, G // 2, body, 0)
    plsc.subcore_barrier()

    rows = NP // 16
    r0 = s * rows
    pltpu.sync_copy(u_sh.at[pl.ds(r0, rows)],
                    u_hbm.at[pl.ds(r0, rows), pl.ds(c * half, half)])


@functools.partial(
    pl.kernel,
    mesh=_mesh,
    out_type=jax.ShapeDtypeStruct((NP, D), jnp.float32),
    scratch_types=[
        pltpu.VMEM_SHARED((NP, 128), jnp.float32),
        pltpu.VMEM((SCH,), jnp.float32),
        pltpu.VMEM((SCH,), jnp.int32),
        pltpu.VMEM((SCH, 128), jnp.float32),
    ],
    compiler_params=_sc_params,
)
def _denom_stage(w_hbm, dst_hbm, zu_hbm, den_hbm, wtab_sh, wbuf, idx_v, wrow_v):
    c = lax.axis_index("c")
    s = lax.axis_index("s")

    @pl.when(s == 0)
    def _():
        pltpu.sync_copy(zu_hbm, wtab_sh)

    pltpu.sync_copy(zu_hbm.at[pl.ds(0, SCH)], wrow_v)
    plsc.subcore_barrier()

    zeros16 = jnp.zeros((16,), jnp.int32)
    iota16 = lax.iota(jnp.int32, 16)
    epw = EP // 32

    def body(g, carry):
        base = (c * 16 + s) * epw + g * SCH
        pltpu.sync_copy(dst_hbm.at[pl.ds(base, SCH)], idx_v)
        pltpu.sync_copy(w_hbm.at[pl.ds(base, SCH)], wbuf)
        for j in range(SCH // 16):
            w16 = wbuf[pl.ds(j * 16, 16)]
            plsc.store_scatter(wrow_v, [iota16 + j * 16, zeros16], w16)
        pltpu.sync_copy(wrow_v, wtab_sh.at[idx_v], add=True)
        return carry

    lax.fori_loop(0, epw // SCH, body, 0)
    plsc.subcore_barrier()

    rows = NP // 16
    r0 = s * rows
    pltpu.sync_copy(wtab_sh.at[pl.ds(r0, rows)],
                    den_hbm.at[pl.ds(r0, rows), pl.ds(c * 128, 128)])


def _final_body(x_ref, u_ref, den_ref, wvt_ref, aux_ref, out_ref):
    bv = aux_ref[5:6, :]
    den = den_ref[:, 0:1] + den_ref[:, 128:129]
    inv = 1.0 / (den + 1e-16)
    aggr = jnp.dot(u_ref[...] * inv, wvt_ref[...],
                   preferred_element_type=jnp.float32) + bv * (den * inv)
    out_ref[...] = x_ref[...] + _gelu(aggr)


def _final_stage(x, u, den, wvt, aux):
    return pl.pallas_call(
        _final_body,
        grid=(N // 400,),
        in_specs=[
            pl.BlockSpec((400, D), lambda i: (i, 0)),
            pl.BlockSpec((400, D), lambda i: (i, 0)),
            pl.BlockSpec((400, D), lambda i: (i, 0)),
            pl.BlockSpec((D, D), lambda i: (0, 0)),
            pl.BlockSpec((8, D), lambda i: (0, 0)),
        ],
        out_specs=pl.BlockSpec((400, D), lambda i: (i, 0)),
        out_shape=jax.ShapeDtypeStruct((N, D), jnp.float32),
    )(x, u, den, wvt, aux)


def kernel(x, edge_index, edge_type, edge_vector, Wt, bt, Wk, bk, Wq, bq,
           Wv, bv, ln_g, ln_b):
    f32 = jnp.float32
    src = edge_index[0]
    dst = edge_index[1]
    pad_e = EP - E
    pad_idx = jnp.full((pad_e,), N, jnp.int32)
    srcp = jnp.concatenate([src, pad_idx])
    dstp = jnp.concatenate([dst, pad_idx])
    evp = jnp.concatenate([edge_vector, jnp.zeros((pad_e, DE), f32)])
    etp = jnp.concatenate([edge_type, jnp.zeros((pad_e, 1), f32)])
    xp = jnp.concatenate([x, jnp.zeros((NP - N, D), f32)])

    zero = jnp.zeros((D,), f32)
    aux = jnp.stack([ln_g, ln_b, bq, bk, bt, bv, zero, zero])
    wt1t = Wt[:, :D].T
    wt2t = Wt[:, D:].T

    t_node, qk, bkq = _node_stage(xp, aux, wt1t, Wq.T, Wk)
    t_src, qk_dst, bkq_dst = _gather_stage(t_node, qk, bkq.reshape(NP),
                                           srcp, dstp)
    m, w = _edge_stage(t_src, qk_dst, bkq_dst.reshape(EP, 1), evp, etp,
                       wt2t, aux)
    zu = jnp.zeros((NP, 128), f32)
    u = _scatter_stage(m, dstp, zu)
    den = _denom_stage(w.reshape(EP), dstp, zu)
    return _final_stage(x, u[:N], den[:N], Wv.T, aux)
